# initial kernel scaffold (unmeasured)
import jax
import jax.numpy as jnp
from jax import lax
from jax.experimental import pallas as pl
from jax.experimental.pallas import tpu as pltpu

N_DEV = 32
M_PER = 128
N_PER = 256


def kernel(x, w_mat):
    m_per, k = x.shape
    _, n = w_mat.shape
    n_per = n // N_DEV

    y = lax.dot_general(
        x, w_mat,
        dimension_numbers=(((1,), (0,)), ((), ())),
        precision=lax.Precision.HIGHEST,
        preferred_element_type=jnp.float32,
    )

    def body(y_ref, out_ref, amax_out_ref,
             recv_buf, amax_buf,
             dsend, drecv, asend, arecv):
        my = lax.axis_index("i")

        local_amax = jnp.max(jnp.abs(y_ref[...]))
        amax_buf[pl.ds(my, 1)] = jnp.full((1, 8, 128), local_amax, jnp.float32)

        out_ref[pl.ds(my * M_PER, M_PER), :] = y_ref[:, pl.ds(my * N_PER, N_PER)]

        sends = []
        for kk in range(1, N_DEV):
            dst = (my + kk) % N_DEV
            d = pltpu.make_async_remote_copy(
                src_ref=y_ref.at[:, pl.ds(dst * N_PER, N_PER)],
                dst_ref=recv_buf.at[my],
                send_sem=dsend.at[kk],
                recv_sem=drecv.at[my],
                device_id=(dst,),
                device_id_type=pl.DeviceIdType.MESH,
            )
            d.start()
            sends.append(d)
            a = pltpu.make_async_remote_copy(
                src_ref=amax_buf.at[my],
                dst_ref=amax_buf.at[my],
                send_sem=asend.at[kk],
                recv_sem=arecv.at[my],
                device_id=(dst,),
                device_id_type=pl.DeviceIdType.MESH,
            )
            a.start()
            sends.append(a)

        for kk in range(1, N_DEV):
            src = (my + (N_DEV - kk)) % N_DEV
            rd = pltpu.make_async_remote_copy(
                src_ref=recv_buf.at[src],
                dst_ref=recv_buf.at[src],
                send_sem=dsend.at[0],
                recv_sem=drecv.at[src],
                device_id=(src,),
                device_id_type=pl.DeviceIdType.MESH,
            )
            rd.wait_recv()
            blk = recv_buf[pl.ds(src, 1)]
            out_ref[pl.ds(src * M_PER, M_PER), :] = blk.reshape(M_PER, N_PER)
            ra = pltpu.make_async_remote_copy(
                src_ref=amax_buf.at[src],
                dst_ref=amax_buf.at[src],
                send_sem=asend.at[0],
                recv_sem=arecv.at[src],
                device_id=(src,),
                device_id_type=pl.DeviceIdType.MESH,
            )
            ra.wait_recv()

        amax_out_ref[0, 0] = jnp.max(amax_buf[...])

        for d in sends:
            d.wait_send()

    out, amax = pl.pallas_call(
        body,
        out_shape=[
            jax.ShapeDtypeStruct((N_DEV * m_per, n_per), jnp.float32),
            jax.ShapeDtypeStruct((1, 1), jnp.float32),
        ],
        in_specs=[pl.BlockSpec(memory_space=pltpu.VMEM)],
        out_specs=[
            pl.BlockSpec(memory_space=pltpu.VMEM),
            pl.BlockSpec(memory_space=pltpu.SMEM),
        ],
        scratch_shapes=[
            pltpu.VMEM((N_DEV, M_PER, N_PER), jnp.float32),
            pltpu.VMEM((N_DEV, 8, 128), jnp.float32),
            pltpu.SemaphoreType.DMA((N_DEV,)),
            pltpu.SemaphoreType.DMA((N_DEV,)),
            pltpu.SemaphoreType.DMA((N_DEV,)),
            pltpu.SemaphoreType.DMA((N_DEV,)),
        ],
        compiler_params=pltpu.CompilerParams(collective_id=0),
    )(y)

    scale = amax[0, 0] / 448.0
    q = (out / scale).astype(jnp.float8_e4m3fn).astype(jnp.float32) * scale
    return q


# baseline (device time: 166287 ns/iter reference)
import jax
import jax.numpy as jnp
from jax import lax
from jax.experimental import pallas as pl
from jax.experimental.pallas import tpu as pltpu

N_DEV = 32
M_PER = 128
N_PER = 256


def kernel(x, w_mat):
    m_per, k = x.shape
    _, n = w_mat.shape
    n_per = n // N_DEV

    y = lax.dot_general(
        x, w_mat,
        dimension_numbers=(((1,), (0,)), ((), ())),
        precision=lax.Precision.HIGHEST,
        preferred_element_type=jnp.float32,
    )

    def body(y_ref, out_ref, amax_out_ref,
             recv_buf, amax_buf,
             dsend, drecv, asend, arecv):
        my = lax.axis_index("i")

        local_amax = jnp.max(jnp.abs(y_ref[...]))
        amax_buf[pl.ds(my, 1)] = jnp.full((1, 8, 128), local_amax, jnp.float32)

        out_ref[pl.ds(my * M_PER, M_PER), :] = y_ref[:, pl.ds(my * N_PER, N_PER)]

        sends = []
        for kk in range(1, N_DEV):
            dst = (my + kk) % N_DEV
            d = pltpu.make_async_remote_copy(
                src_ref=y_ref.at[:, pl.ds(dst * N_PER, N_PER)],
                dst_ref=recv_buf.at[my],
                send_sem=dsend.at[kk],
                recv_sem=drecv.at[my],
                device_id=(dst,),
                device_id_type=pl.DeviceIdType.MESH,
            )
            d.start()
            sends.append(d)
            a = pltpu.make_async_remote_copy(
                src_ref=amax_buf.at[my],
                dst_ref=amax_buf.at[my],
                send_sem=asend.at[kk],
                recv_sem=arecv.at[my],
                device_id=(dst,),
                device_id_type=pl.DeviceIdType.MESH,
            )
            a.start()
            sends.append(a)

        for kk in range(1, N_DEV):
            src = (my + (N_DEV - kk)) % N_DEV
            rd = pltpu.make_async_remote_copy(
                src_ref=recv_buf.at[src],
                dst_ref=recv_buf.at[src],
                send_sem=dsend.at[0],
                recv_sem=drecv.at[src],
                device_id=(src,),
                device_id_type=pl.DeviceIdType.MESH,
            )
            rd.wait_recv()
            blk = recv_buf[pl.ds(src, 1)]
            out_ref[pl.ds(src * M_PER, M_PER), :] = blk.reshape(M_PER, N_PER)
            ra = pltpu.make_async_remote_copy(
                src_ref=amax_buf.at[src],
                dst_ref=amax_buf.at[src],
                send_sem=asend.at[0],
                recv_sem=arecv.at[src],
                device_id=(src,),
                device_id_type=pl.DeviceIdType.MESH,
            )
            ra.wait_recv()

        amax_out_ref[0, 0] = jnp.max(amax_buf[...])

        for d in sends:
            d.wait_send()

    out, amax = pl.pallas_call(
        body,
        out_shape=[
            jax.ShapeDtypeStruct((N_DEV * m_per, n_per), jnp.float32),
            jax.ShapeDtypeStruct((1, 1), jnp.float32),
        ],
        in_specs=[pl.BlockSpec(memory_space=pltpu.VMEM)],
        out_specs=[
            pl.BlockSpec(memory_space=pltpu.VMEM),
            pl.BlockSpec(memory_space=pltpu.SMEM),
        ],
        scratch_shapes=[
            pltpu.VMEM((N_DEV, M_PER, N_PER), jnp.float32),
            pltpu.VMEM((N_DEV, 8, 128), jnp.float32),
            pltpu.SemaphoreType.DMA((N_DEV,)),
            pltpu.SemaphoreType.DMA((N_DEV,)),
            pltpu.SemaphoreType.DMA((N_DEV,)),
            pltpu.SemaphoreType.DMA((N_DEV,)),
        ],
    )(y)

    scale = amax[0, 0] / 448.0
    a = jnp.abs(out / scale)
    _, e = jnp.frexp(a)
    step = jnp.where(a >= 2.0 ** -6,
                     jnp.ldexp(jnp.float32(1.0), e - 4),
                     jnp.float32(2.0 ** -9))
    snapped = jnp.minimum(jnp.round(a / step) * step, 448.0)
    return jnp.sign(out) * snapped * scale


# device time: 69596 ns/iter; 2.3893x vs baseline; 2.3893x over previous
import jax
import jax.numpy as jnp
from jax import lax
from jax.experimental import pallas as pl
from jax.experimental.pallas import tpu as pltpu

N_DEV = 32
M_PER = 128
N_PER = 256
W_SLOTS = 4
COMM_DTYPE = jnp.bfloat16


def _snap_e4m3(y, scale, inv_scale):
    a = jnp.abs(y) * inv_scale
    xi = lax.bitcast_convert_type(a, jnp.int32)
    rb = ((xi >> 20) & 1) + jnp.int32(0x7FFFF)
    xr = lax.bitcast_convert_type((xi + rb) & jnp.int32(-1048576), jnp.float32)
    sub = (a * 512.0 + 16777216.0 - 16777216.0) * (1.0 / 512.0)
    snapped = jnp.minimum(jnp.where(a >= 2.0 ** -6, xr, sub), 448.0)
    return jnp.where(y < 0, -snapped, snapped) * scale


def kernel(x, w_mat):
    m_per, k = x.shape
    _, n = w_mat.shape

    def body(x_ref, w_hbm, out_ref,
             wbuf, sendbuf, recv_buf, amax_buf,
             wsems, dsend, drecv, asend, arecv):
        my = lax.axis_index("i")

        def w_copy(kk):
            dst = (my + kk) % N_DEV
            return pltpu.make_async_copy(
                w_hbm.at[:, pl.ds(dst * N_PER, N_PER)],
                wbuf.at[kk % W_SLOTS],
                wsems.at[kk % W_SLOTS],
            )

        for kk in range(min(W_SLOTS, N_DEV)):
            w_copy(kk).start()

        running_amax = jnp.float32(0.0)
        sends = []
        for kk in range(N_DEV):
            dst = (my + kk) % N_DEV
            w_copy(kk).wait()
            y_b = lax.dot_general(
                x_ref[...], wbuf[kk % W_SLOTS],
                dimension_numbers=(((1,), (0,)), ((), ())),
                precision=lax.Precision.DEFAULT,
                preferred_element_type=jnp.float32,
            )
            if kk + W_SLOTS < N_DEV:
                w_copy(kk + W_SLOTS).start()
            running_amax = jnp.maximum(running_amax, jnp.max(jnp.abs(y_b)))
            yh = y_b.astype(COMM_DTYPE).reshape(1, M_PER, N_PER)
            if kk == 0:
                recv_buf[pl.ds(my, 1)] = yh
            else:
                sendbuf[pl.ds(dst, 1)] = yh
                d = pltpu.make_async_remote_copy(
                    src_ref=sendbuf.at[dst],
                    dst_ref=recv_buf.at[my],
                    send_sem=dsend.at[kk],
                    recv_sem=drecv.at[my],
                    device_id=(dst,),
                    device_id_type=pl.DeviceIdType.MESH,
                )
                d.start()
                sends.append(d)

        amax_buf[pl.ds(my, 1)] = jnp.full((1, 8, 128), running_amax, jnp.float32)
        for kk in range(1, N_DEV):
            dst = (my + kk) % N_DEV
            a = pltpu.make_async_remote_copy(
                src_ref=amax_buf.at[my],
                dst_ref=amax_buf.at[my],
                send_sem=asend.at[kk],
                recv_sem=arecv.at[my],
                device_id=(dst,),
                device_id_type=pl.DeviceIdType.MESH,
            )
            a.start()
            sends.append(a)
        for kk in range(1, N_DEV):
            src = (my + (N_DEV - kk)) % N_DEV
            pltpu.make_async_remote_copy(
                src_ref=amax_buf.at[src],
                dst_ref=amax_buf.at[src],
                send_sem=asend.at[0],
                recv_sem=arecv.at[src],
                device_id=(src,),
                device_id_type=pl.DeviceIdType.MESH,
            ).wait_recv()

        g_amax = jnp.max(amax_buf[...])
        scale = g_amax / 448.0
        inv_scale = 448.0 / g_amax

        own = recv_buf[pl.ds(my, 1)].reshape(M_PER, N_PER).astype(jnp.float32)
        out_ref[pl.ds(my * M_PER, M_PER), :] = _snap_e4m3(own, scale, inv_scale)
        for kk in range(1, N_DEV):
            src = (my + (N_DEV - kk)) % N_DEV
            pltpu.make_async_remote_copy(
                src_ref=recv_buf.at[src],
                dst_ref=recv_buf.at[src],
                send_sem=dsend.at[0],
                recv_sem=drecv.at[src],
                device_id=(src,),
                device_id_type=pl.DeviceIdType.MESH,
            ).wait_recv()
            blk = recv_buf[pl.ds(src, 1)].reshape(M_PER, N_PER).astype(jnp.float32)
            out_ref[pl.ds(src * M_PER, M_PER), :] = _snap_e4m3(blk, scale, inv_scale)

        for d in sends:
            d.wait_send()

    return pl.pallas_call(
        body,
        out_shape=jax.ShapeDtypeStruct((N_DEV * m_per, N_PER), jnp.float32),
        in_specs=[
            pl.BlockSpec(memory_space=pltpu.VMEM),
            pl.BlockSpec(memory_space=pl.ANY),
        ],
        out_specs=pl.BlockSpec(memory_space=pltpu.VMEM),
        scratch_shapes=[
            pltpu.VMEM((W_SLOTS, k, N_PER), jnp.float32),
            pltpu.VMEM((N_DEV, M_PER, N_PER), COMM_DTYPE),
            pltpu.VMEM((N_DEV, M_PER, N_PER), COMM_DTYPE),
            pltpu.VMEM((N_DEV, 8, 128), jnp.float32),
            pltpu.SemaphoreType.DMA((W_SLOTS,)),
            pltpu.SemaphoreType.DMA((N_DEV,)),
            pltpu.SemaphoreType.DMA((N_DEV,)),
            pltpu.SemaphoreType.DMA((N_DEV,)),
            pltpu.SemaphoreType.DMA((N_DEV,)),
        ],
    )(x, w_mat)
